# baseline (speedup 1.0000x reference)
import functools

import jax
import jax.numpy as jnp
from jax import lax
from jax.experimental import pallas as pl
from jax.experimental.pallas import tpu as pltpu
from jax.experimental.pallas import tpu_sc as plsc

NUM_ROWS = 100000
DIM = 64
RANK = 8

NC = 2
NS = 16
NW = NC * NS
IDX_TOTAL = 204800
CHUNK = 512
NCHUNKS = IDX_TOTAL // CHUNK
ITERS = (NCHUNKS + NW - 1) // NW

FUSE_BLK = 16384
FUSE_H = FUSE_BLK // 2
NBLK = (NUM_ROWS + FUSE_BLK - 1) // FUSE_BLK
NPAD = NBLK * FUSE_BLK


def _fuse_body(wt_ref, bt_ref, a_ref, t2_ref):
    tt = wt_ref[...] + jnp.dot(a_ref[...].T, bt_ref[...],
                               preferred_element_type=jnp.float32)
    u = jnp.concatenate([tt[:, :FUSE_H], tt[:, FUSE_H:]],
                        axis=0)
    t2_ref[...] = jnp.transpose(u, (1, 0)).astype(jnp.bfloat16)


def _fuse_table(Wt, A, Bt):
    return pl.pallas_call(
        _fuse_body,
        grid=(NBLK,),
        in_specs=[
            pl.BlockSpec((DIM, FUSE_BLK), lambda i: (0, i)),
            pl.BlockSpec((RANK, FUSE_BLK), lambda i: (0, i)),
            pl.BlockSpec((RANK, DIM), lambda i: (0, 0)),
        ],
        out_specs=pl.BlockSpec((FUSE_BLK // 2, 2 * DIM), lambda i: (i, 0)),
        out_shape=jax.ShapeDtypeStruct((NPAD // 2, 2 * DIM), jnp.bfloat16),
    )(Wt, Bt, A)


def _sc_gather(table, idx):
    mesh = plsc.VectorSubcoreMesh(core_axis_name="c", subcore_axis_name="s")
    half = CHUNK // 2

    @functools.partial(
        pl.kernel,
        mesh=mesh,
        compiler_params=pltpu.CompilerParams(use_tc_tiling_on_sc=False),
        out_type=jax.ShapeDtypeStruct((NCHUNKS, half, 2 * DIM), jnp.bfloat16),
        scratch_types=[
            pltpu.VMEM((2 * CHUNK,), jnp.int32),
            pltpu.VMEM((2, CHUNK, DIM), jnp.bfloat16),
            pltpu.SemaphoreType.DMA,
            pltpu.SemaphoreType.DMA,
        ],
    )
    def k(table_hbm, idx_hbm, out_hbm, raw_v, rows_v, gsem, wsem):
        wid = lax.axis_index("s") * NC + lax.axis_index("c")

        def chunk_of(it):
            return jnp.minimum(wid + NW * it, NCHUNKS - 1)

        def prep_idx(it):
            t = chunk_of(it)
            buf = lax.rem(it, 2)
            base = 2048 * (t // 4) + half * (t % 4)
            pltpu.sync_copy(idx_hbm.at[pl.ds(base, half)],
                            raw_v.at[pl.ds(CHUNK * buf, half)])
            pltpu.sync_copy(idx_hbm.at[pl.ds(base + 1024, half)],
                            raw_v.at[pl.ds(CHUNK * buf + half, half)])

        def fire(it):
            buf = lax.rem(it, 2)
            descs = []
            for s in range(4):
                descs.append(pltpu.async_copy(
                    table_hbm.at[raw_v.at[pl.ds(CHUNK * buf + 128 * s, 128)]],
                    rows_v.at[buf].at[pl.ds(128 * s, 128)], gsem))
            return descs

        def write(it):
            buf = lax.rem(it, 2)
            t = chunk_of(it)
            for h in range(2):
                pltpu.async_copy(
                    rows_v.at[buf].at[pl.ds(half * h, half)],
                    out_hbm.at[t].at[:, pl.ds(DIM * h, DIM)], wsem)

        def drain_write(it):
            buf = lax.rem(it, 2)
            for h in range(2):
                pltpu.make_async_copy(
                    table_hbm.at[pl.ds(0, half)],
                    rows_v.at[buf].at[pl.ds(half * h, half)], wsem).wait()

        prep_idx(0)
        d0 = fire(0)
        prep_idx(1)
        for d in d0:
            d.wait()
        write(0)

        def body(it, carry):
            descs = fire(it)
            prep_idx(it + 1)
            for d in descs:
                d.wait()
            drain_write(it - 1)
            write(it)
            return carry

        lax.fori_loop(1, ITERS - 1, body, 0)

        it = ITERS - 1
        descs = fire(it)
        for d in descs:
            d.wait()
        drain_write(it - 1)
        write(it)
        drain_write(it)

    return k(table, idx)


def _xpose_body(g_ref, z_ref):
    for r in range(5):
        g = g_ref[r].astype(jnp.float32)
        gt0 = jnp.transpose(g[:1024], (1, 0))
        gt1 = jnp.transpose(g[1024:], (1, 0))
        z_ref[r] = jnp.concatenate(
            [gt0[:DIM], gt0[DIM:], gt1[:DIM], gt1[DIM:]], axis=1)


def _xpose(gp):
    return pl.pallas_call(
        _xpose_body,
        grid=(10,),
        in_specs=[pl.BlockSpec((5, 2048, 128), lambda j: (j, 0, 0))],
        out_specs=pl.BlockSpec((5, DIM, 4096), lambda j: (j, 0, 0)),
        out_shape=jax.ShapeDtypeStruct((50, DIM, 4096), jnp.float32),
    )(gp)


def kernel(x, W, A, B):
    t2 = _fuse_table(W.T, A, B.T)
    table = t2.reshape(NPAD, DIM)
    xt = x.T.reshape(IDX_TOTAL)
    idx = ((xt & ~jnp.int32(FUSE_BLK - 1)) | ((xt & (FUSE_H - 1)) << 1)
           | ((xt // FUSE_H) & 1))
    out = _sc_gather(table, idx)
    gp = out.reshape(50, 2048, 2 * DIM)
    z = _xpose(gp)
    return z.transpose(2, 0, 1)


# speedup vs baseline: 2.1349x; 2.1349x over previous
import functools

import jax
import jax.numpy as jnp
from jax import lax
from jax.experimental import pallas as pl
from jax.experimental.pallas import tpu as pltpu
from jax.experimental.pallas import tpu_sc as plsc

NUM_ROWS = 100000
DIM = 64
RANK = 8

NC = 2
NS = 16
NW = NC * NS
IDX_TOTAL = 204800
CHUNK = 512
NCHUNKS = IDX_TOTAL // CHUNK
ITERS = (NCHUNKS + NW - 1) // NW

FUSE_BLK = 16384
FUSE_H = FUSE_BLK // 2
NBLK = (NUM_ROWS + FUSE_BLK - 1) // FUSE_BLK
NPAD = NBLK * FUSE_BLK


def _fuse_body(wt_ref, bt_ref, a_ref, t2_ref):
    tt = wt_ref[...] + jnp.dot(a_ref[...].T, bt_ref[...],
                               preferred_element_type=jnp.float32)
    u = jnp.concatenate([tt[:, :FUSE_H], tt[:, FUSE_H:]],
                        axis=0)
    t2_ref[...] = jnp.transpose(u, (1, 0))


def _fuse_table(Wt, A, Bt):
    return pl.pallas_call(
        _fuse_body,
        grid=(NBLK,),
        in_specs=[
            pl.BlockSpec((DIM, FUSE_BLK), lambda i: (0, i)),
            pl.BlockSpec((RANK, FUSE_BLK), lambda i: (0, i)),
            pl.BlockSpec((RANK, DIM), lambda i: (0, 0)),
        ],
        out_specs=pl.BlockSpec((FUSE_BLK // 2, 2 * DIM), lambda i: (i, 0)),
        out_shape=jax.ShapeDtypeStruct((NPAD // 2, 2 * DIM), jnp.float32),
    )(Wt, Bt, A)


def _sc_gather(table, idx):
    mesh = plsc.VectorSubcoreMesh(core_axis_name="c", subcore_axis_name="s")
    half = CHUNK // 2

    @functools.partial(
        pl.kernel,
        mesh=mesh,
        compiler_params=pltpu.CompilerParams(use_tc_tiling_on_sc=False),
        out_type=jax.ShapeDtypeStruct((NCHUNKS, half, 2 * DIM), jnp.float32),
        scratch_types=[
            pltpu.VMEM((2 * CHUNK,), jnp.int32),
            pltpu.VMEM((2, CHUNK, DIM), jnp.float32),
            pltpu.SemaphoreType.DMA,
            pltpu.SemaphoreType.DMA,
        ],
    )
    def k(table_hbm, idx_hbm, out_hbm, raw_v, rows_v, gsem, wsem):
        wid = lax.axis_index("s") * NC + lax.axis_index("c")

        def chunk_of(it):
            return jnp.minimum(wid + NW * it, NCHUNKS - 1)

        def prep_idx(it):
            t = chunk_of(it)
            buf = lax.rem(it, 2)
            base = 2048 * (t // 4) + half * (t % 4)
            pltpu.sync_copy(idx_hbm.at[pl.ds(base, half)],
                            raw_v.at[pl.ds(CHUNK * buf, half)])
            pltpu.sync_copy(idx_hbm.at[pl.ds(base + 1024, half)],
                            raw_v.at[pl.ds(CHUNK * buf + half, half)])

        def fire(it):
            buf = lax.rem(it, 2)
            descs = []
            for s in range(4):
                descs.append(pltpu.async_copy(
                    table_hbm.at[raw_v.at[pl.ds(CHUNK * buf + 128 * s, 128)]],
                    rows_v.at[buf].at[pl.ds(128 * s, 128)], gsem))
            return descs

        def write(it):
            buf = lax.rem(it, 2)
            t = chunk_of(it)
            for h in range(2):
                pltpu.async_copy(
                    rows_v.at[buf].at[pl.ds(half * h, half)],
                    out_hbm.at[t].at[:, pl.ds(DIM * h, DIM)], wsem)

        def drain_write(it):
            buf = lax.rem(it, 2)
            for h in range(2):
                pltpu.make_async_copy(
                    table_hbm.at[pl.ds(0, half)],
                    rows_v.at[buf].at[pl.ds(half * h, half)], wsem).wait()

        prep_idx(0)
        d0 = fire(0)
        prep_idx(1)
        for d in d0:
            d.wait()
        write(0)

        def body(it, carry):
            descs = fire(it)
            prep_idx(it + 1)
            for d in descs:
                d.wait()
            drain_write(it - 1)
            write(it)
            return carry

        lax.fori_loop(1, ITERS - 1, body, 0)

        it = ITERS - 1
        descs = fire(it)
        for d in descs:
            d.wait()
        drain_write(it - 1)
        write(it)
        drain_write(it)

    return k(table, idx)


def _xpose_body(g_ref, z_ref):
    for r in range(5):
        g = g_ref[r]
        gt0 = jnp.transpose(g[:1024], (1, 0))
        gt1 = jnp.transpose(g[1024:], (1, 0))
        z_ref[r] = jnp.concatenate(
            [gt0[:DIM], gt0[DIM:], gt1[:DIM], gt1[DIM:]], axis=1)


def _xpose(gp):
    return pl.pallas_call(
        _xpose_body,
        grid=(10,),
        in_specs=[pl.BlockSpec((5, 2048, 128), lambda j: (j, 0, 0))],
        out_specs=pl.BlockSpec((5, DIM, 4096), lambda j: (j, 0, 0)),
        out_shape=jax.ShapeDtypeStruct((50, DIM, 4096), jnp.float32),
    )(gp)


def kernel(x, W, A, B):
    t2 = _fuse_table(W.T, A, B.T)
    table = t2.reshape(NPAD, DIM)
    xt = x.T.reshape(IDX_TOTAL)
    idx = ((xt & ~jnp.int32(FUSE_BLK - 1)) | ((xt & (FUSE_H - 1)) << 1)
           | ((xt // FUSE_H) & 1))
    out = _sc_gather(table, idx)
    gp = out.reshape(50, 2048, 2 * DIM)
    z = _xpose(gp)
    return z.transpose(2, 0, 1)
